# baseline (speedup 1.0000x reference)
import jax
import jax.numpy as jnp
from jax.experimental import pallas as pl
from jax.experimental.pallas import tpu as pltpu

_CHANNELS = 64
_LANES = 128


def _body(idx_ref, tablet_ref, out_ref):
    r = idx_ref[0] % _LANES
    onehot = (
        jax.lax.broadcasted_iota(jnp.int32, (1, _LANES), 1) == r
    ).astype(jnp.float32)
    out_ref[...] = jax.lax.dot_general(
        onehot,
        tablet_ref[...],
        (((1,), (1,)), ((), ())),
        preferred_element_type=jnp.float32,
    )


def kernel(t, pos_enc_weight):
    idx = jnp.asarray(t, dtype=jnp.int32).reshape(1)
    tablet = pos_enc_weight.T
    grid_spec = pltpu.PrefetchScalarGridSpec(
        num_scalar_prefetch=1,
        grid=(1,),
        in_specs=[
            pl.BlockSpec(
                (_CHANNELS, _LANES), lambda i, idx_ref: (0, idx_ref[0] // _LANES)
            )
        ],
        out_specs=pl.BlockSpec((1, _CHANNELS), lambda i, idx_ref: (0, 0)),
    )
    return pl.pallas_call(
        _body,
        grid_spec=grid_spec,
        out_shape=jax.ShapeDtypeStruct((1, _CHANNELS), jnp.float32),
    )(idx, tablet)
